# baseline (speedup 1.0000x reference)
import functools

import jax
import jax.numpy as jnp
from jax import lax
from jax.experimental import pallas as pl
from jax.experimental.pallas import tpu as pltpu
from jax.experimental.pallas import tpu_sc as plsc

NC = 2
NS = 16
CHUNK = 128


def _sc_mesh():
    return plsc.VectorSubcoreMesh(
        core_axis_name="c", subcore_axis_name="s", num_cores=NC, num_subcores=NS
    )


def _fill_vmem(ref, value):
    v = jnp.full((16,), value, jnp.float32)

    @pl.loop(0, ref.shape[0])
    def _(i):
        @pl.loop(0, ref.shape[1], step=16)
        def _(j):
            ref[i, pl.ds(j, 16)] = v


NBUF = 4
NROWS = 2


def _sc_aggregate(h, src2, dst2, n_pad):
    n, d = h.shape
    nchunks = src2.shape[0]
    cpt = nchunks // (NC * NS)
    pad_per_sub = n_pad // NS
    zrows = 8

    @functools.partial(
        pl.kernel,
        out_type=jax.ShapeDtypeStruct((NC, n_pad, d), jnp.float32),
        mesh=_sc_mesh(),
        scratch_types=[
            pltpu.VMEM((NBUF, CHUNK), jnp.int32),
            pltpu.VMEM((NBUF, CHUNK), jnp.int32),
            [pltpu.VMEM((CHUNK, d), jnp.float32) for _ in range(NROWS)],
            pltpu.VMEM((zrows, d), jnp.float32),
            pltpu.VMEM_SHARED((n_pad, d), jnp.float32),
            [pltpu.SemaphoreType.DMA for _ in range(NBUF)],
            [pltpu.SemaphoreType.DMA for _ in range(NROWS)],
        ],
    )
    def agg(h_hbm, src_hbm, dst_hbm, out_hbm, src_i, dst_i, rows_v, zero_v,
            acc_sh, sem_i, sem_g):
        cid = lax.axis_index("c")
        sid = lax.axis_index("s")
        wid = sid * NC + cid
        row0 = wid * cpt

        def idx_load(k, ib):
            pltpu.async_copy(src_hbm.at[row0 + k], src_i.at[ib], sem_i[ib])
            pltpu.async_copy(dst_hbm.at[row0 + k], dst_i.at[ib], sem_i[ib])

        def idx_wait(k, ib):
            pltpu.make_async_copy(src_hbm.at[row0 + k], src_i.at[ib],
                                  sem_i[ib]).wait()
            pltpu.make_async_copy(dst_hbm.at[row0 + k], dst_i.at[ib],
                                  sem_i[ib]).wait()

        def gather_start(ib, rb):
            pltpu.async_copy(h_hbm.at[src_i.at[ib]], rows_v[rb], sem_g[rb])

        def gather_wait(ib, rb):
            pltpu.make_async_copy(h_hbm.at[src_i.at[ib]], rows_v[rb],
                                  sem_g[rb]).wait()

        _fill_vmem(zero_v, 0.0)
        zbase = sid * pad_per_sub

        @pl.loop(0, pad_per_sub, step=zrows)
        def _(r):
            pltpu.sync_copy(zero_v, acc_sh.at[pl.ds(zbase + r, zrows)])

        plsc.subcore_barrier()

        for k in range(NBUF):
            idx_load(k, k)
        for k in range(NROWS):
            idx_wait(k, k)
            gather_start(k, k)

        @pl.loop(0, cpt, step=NBUF)
        def _(j):
            for u in range(NBUF):
                ib = u
                rb = u % NROWS
                k = j + u
                gather_wait(ib, rb)
                pltpu.sync_copy(rows_v[rb], acc_sh.at[dst_i.at[ib]], add=True)

                @pl.when(k + NBUF < cpt)
                def _():
                    idx_load(k + NBUF, ib)

                @pl.when(k + NROWS < cpt)
                def _():
                    idx_wait(k + NROWS, (u + NROWS) % NBUF)
                    gather_start((u + NROWS) % NBUF, rb)

        plsc.subcore_barrier()

        pltpu.sync_copy(acc_sh.at[pl.ds(zbase, pad_per_sub)],
                        out_hbm.at[cid, pl.ds(zbase, pad_per_sub)])

    return agg(h, src2, dst2)


def _sc_count(dst2, n, n_pad):
    nchunks = dst2.shape[0]
    cpt = nchunks // (NC * NS)
    w = 128
    pad_per_sub = n_pad // NS
    zrows = 64

    @functools.partial(
        pl.kernel,
        out_type=jax.ShapeDtypeStruct((NC, n_pad, w), jnp.float32),
        mesh=_sc_mesh(),
        scratch_types=[
            pltpu.VMEM((cpt, CHUNK), jnp.int32),
            pltpu.VMEM((CHUNK, w), jnp.float32),
            pltpu.VMEM((zrows, w), jnp.float32),
            pltpu.VMEM_SHARED((n_pad, w), jnp.float32),
        ],
    )
    def count(dst_hbm, out_hbm, dst_v, ones_v, zero_v, cnt_sh):
        cid = lax.axis_index("c")
        sid = lax.axis_index("s")
        wid = sid * NC + cid

        row0 = wid * cpt
        pltpu.sync_copy(dst_hbm.at[pl.ds(row0, cpt)], dst_v)

        _fill_vmem(ones_v, 1.0)
        _fill_vmem(zero_v, 0.0)
        zbase = sid * pad_per_sub

        @pl.loop(0, pad_per_sub, step=zrows)
        def _(r):
            pltpu.sync_copy(zero_v, cnt_sh.at[pl.ds(zbase + r, zrows)])

        plsc.subcore_barrier()

        @pl.loop(0, cpt)
        def _(j):
            pltpu.sync_copy(ones_v, cnt_sh.at[dst_v.at[j]], add=True)

        plsc.subcore_barrier()

        pltpu.sync_copy(cnt_sh.at[pl.ds(zbase, pad_per_sub)],
                        out_hbm.at[cid, pl.ds(zbase, pad_per_sub)])

    return count(dst2)


def _tc_combine(h, p0, p1, d0, d1, wr, wn, bi, ai):
    n, d = h.shape
    bm = 1000

    def body(h_ref, p0_ref, p1_ref, d0_ref, d1_ref, wr_ref, wn_ref, b_ref,
             a_ref, o_ref):
        hh = h_ref[...]
        agg = p0_ref[...] + p1_ref[...]
        deg = jnp.maximum(d0_ref[...] + d1_ref[...], 1.0)
        mean = agg / deg
        v = (jnp.dot(hh, wr_ref[...], preferred_element_type=jnp.float32)
             + jnp.dot(mean, wn_ref[...], preferred_element_type=jnp.float32)
             + b_ref[...])
        o_ref[...] = hh + jnp.maximum(v, 0.0) + a_ref[...] * jnp.minimum(v, 0.0)

    return pl.pallas_call(
        body,
        grid=(n // bm,),
        in_specs=[
            pl.BlockSpec((bm, d), lambda i: (i, 0)),
            pl.BlockSpec((bm, d), lambda i: (i, 0)),
            pl.BlockSpec((bm, d), lambda i: (i, 0)),
            pl.BlockSpec((bm, 1), lambda i: (i, 0)),
            pl.BlockSpec((bm, 1), lambda i: (i, 0)),
            pl.BlockSpec((d, d), lambda i: (0, 0)),
            pl.BlockSpec((d, d), lambda i: (0, 0)),
            pl.BlockSpec((1, d), lambda i: (0, 0)),
            pl.BlockSpec((1, d), lambda i: (0, 0)),
        ],
        out_specs=pl.BlockSpec((bm, d), lambda i: (i, 0)),
        out_shape=jax.ShapeDtypeStruct((n, d), jnp.float32),
    )(h, p0, p1, d0, d1, wr, wn, bi, ai)


def kernel(x, edge_index, W_root, W_nei, b, prelu_a):
    n, d = x.shape
    src = edge_index[0]
    dst = edge_index[1]
    e = src.shape[0]
    nconv = W_root.shape[0]

    step = NS * 64
    n_pad = ((n + step - 1) // step) * step

    gran = CHUNK * NC * NS * NBUF
    e_pad = ((e + gran - 1) // gran) * gran
    if e_pad != e:
        src = jnp.concatenate([src, jnp.zeros((e_pad - e,), jnp.int32)])
        dst = jnp.concatenate([dst, jnp.full((e_pad - e,), n, jnp.int32)])
    src2 = src.reshape(e_pad // CHUNK, CHUNK)
    dst2 = dst.reshape(e_pad // CHUNK, CHUNK)

    cnt = _sc_count(dst2, n, n_pad)
    d0 = cnt[0, :n, :1]
    d1 = cnt[1, :n, :1]

    h = x
    for i in range(nconv):
        p = _sc_aggregate(h, src2, dst2, n_pad)
        if i < nconv - 1:
            ai = jnp.full((1, d), prelu_a[i], jnp.float32)
        else:
            ai = jnp.ones((1, d), jnp.float32)
        h = _tc_combine(h, p[0, :n], p[1, :n], d0, d1, W_root[i], W_nei[i],
                        b[i].reshape(1, d), ai)
    return h
